# Initial kernel scaffold; baseline (speedup 1.0000x reference)
import jax
import jax.numpy as jnp
from jax.experimental import pallas as pl


def kernel(positions, numbers):
    raise NotImplementedError("write your pallas kernel here")


# speedup vs baseline: 4.9001x; 4.9001x over previous
import functools

import jax
import jax.numpy as jnp
from jax.experimental import pallas as pl

_K = 16
_ROWS = 256


def _knn_tile_kernel(n_real, qpos_ref, kpos_ref, idx_ref, dst_ref, dist_ref):
    rows = qpos_ref.shape[0]
    npad = kpos_ref.shape[1]

    xq = qpos_ref[:, 0:1]
    yq = qpos_ref[:, 1:2]
    zq = qpos_ref[:, 2:3]
    xk = kpos_ref[0:1, :]
    yk = kpos_ref[1:2, :]
    zk = kpos_ref[2:3, :]

    sqq = (xq * xq + yq * yq) + zq * zq
    sqk = (xk * xk + yk * yk) + zk * zk
    dot = jax.lax.dot_general(
        qpos_ref[...], kpos_ref[...], (((1,), (0,)), ((), ())),
        preferred_element_type=jnp.float32)
    d2 = (sqq + sqk) - 2.0 * dot

    col = jax.lax.broadcasted_iota(jnp.int32, (rows, npad), 1)
    inf = jnp.float32(jnp.inf)
    d2 = jnp.where(col >= n_real, inf, d2)

    big = jnp.int32(npad)
    vals = []
    idxs = []
    for _ in range(_K):
        m = jnp.min(d2, axis=1, keepdims=True)
        sel = jnp.where(d2 == m, col, big)
        idx = jnp.min(sel, axis=1, keepdims=True)
        vals.append(m)
        idxs.append(idx)
        d2 = jnp.where(col == idx, inf, d2)

    v = jnp.concatenate(vals, axis=1)
    i = jnp.concatenate(idxs, axis=1)
    idx_ref[...] = i
    dist_ref[...] = jnp.sqrt(jnp.maximum(v, 0.0))
    row0 = pl.program_id(0) * rows
    dst_ref[...] = row0 + jax.lax.broadcasted_iota(jnp.int32, (rows, _K), 0)


def kernel(positions, numbers):
    n = positions.shape[0]
    npad = ((n + _ROWS - 1) // _ROWS) * _ROWS
    grid = npad // _ROWS

    pos_pad = jnp.zeros((npad, 128), jnp.float32).at[:n, :3].set(positions)
    kpos = jnp.zeros((128, npad), jnp.float32).at[:3, :n].set(positions.T)

    idx, dst, dists = pl.pallas_call(
        functools.partial(_knn_tile_kernel, n),
        grid=(grid,),
        in_specs=[
            pl.BlockSpec((_ROWS, 128), lambda i: (i, 0)),
            pl.BlockSpec((128, npad), lambda i: (0, 0)),
        ],
        out_specs=[
            pl.BlockSpec((_ROWS, _K), lambda i: (i, 0)),
            pl.BlockSpec((_ROWS, _K), lambda i: (i, 0)),
            pl.BlockSpec((_ROWS, _K), lambda i: (i, 0)),
        ],
        out_shape=[
            jax.ShapeDtypeStruct((npad, _K), jnp.int32),
            jax.ShapeDtypeStruct((npad, _K), jnp.int32),
            jax.ShapeDtypeStruct((npad, _K), jnp.float32),
        ],
    )(pos_pad, kpos)

    src = idx[:n].reshape(-1)
    dst = dst[:n].reshape(-1)
    return (src, dst, numbers, dists[:n])


# speedup vs baseline: 13.6279x; 2.7812x over previous
import functools

import jax
import jax.numpy as jnp
from jax.experimental import pallas as pl

_K = 16
_ROWS = 256
_CHUNK = 128
_BATCH = 16


def _batcher_pairs(n):
    pairs = []

    def oddeven_merge(lo, hi, r):
        step = r * 2
        if step < hi - lo:
            oddeven_merge(lo, hi, step)
            oddeven_merge(lo + r, hi, step)
            for i in range(lo + r, hi - r, step):
                pairs.append((i, i + r))
        else:
            pairs.append((lo, lo + r))

    def sort(lo, hi):
        if hi - lo >= 1:
            mid = lo + (hi - lo) // 2
            sort(lo, mid)
            sort(mid + 1, hi)
            oddeven_merge(lo, hi, 1)

    sort(0, n - 1)
    return pairs


def _ce(v, c, i, j):
    m = v[i] <= v[j]
    lo_v = jnp.where(m, v[i], v[j])
    hi_v = jnp.where(m, v[j], v[i])
    lo_c = jnp.where(m, c[i], c[j])
    hi_c = jnp.where(m, c[j], c[i])
    v[i], v[j] = lo_v, hi_v
    c[i], c[j] = lo_c, hi_c


_SORT16 = _batcher_pairs(16)


def _sort_batch(bv, bc):
    for i, j in _SORT16:
        _ce(bv, bc, i, j)
    return bv, bc


def _merge_keep_low(rv, rc, bv, bc):
    n = len(rv)
    v = []
    c = []
    for i in range(n):
        m = rv[i] <= bv[n - 1 - i]
        v.append(jnp.where(m, rv[i], bv[n - 1 - i]))
        c.append(jnp.where(m, rc[i], bc[n - 1 - i]))
    d = n // 2
    while d >= 1:
        for i in range(n):
            if i & d == 0:
                _ce(v, c, i, i + d)
        d //= 2
    return v, c


def _knn_tile_kernel(n_real, qpos_ref, kpos_ref, idx_ref, dst_ref, dist_ref):
    rows = qpos_ref.shape[0]
    npad = kpos_ref.shape[1]
    nchunks = npad // _CHUNK

    xq = qpos_ref[:, 0:1]
    yq = qpos_ref[:, 1:2]
    zq = qpos_ref[:, 2:3]
    xk = kpos_ref[0:1, :]
    yk = kpos_ref[1:2, :]
    zk = kpos_ref[2:3, :]

    inf = jnp.float32(jnp.inf)
    sqq = (xq * xq + yq * yq) + zq * zq
    sqk = (xk * xk + yk * yk) + zk * zk
    kcol = jax.lax.broadcasted_iota(jnp.int32, (1, npad), 1)
    sqk = jnp.where(kcol >= n_real, inf, sqk)
    dot = jax.lax.dot_general(
        qpos_ref[...], kpos_ref[...], (((1,), (0,)), ((), ())),
        preferred_element_type=jnp.float32)
    d2 = (sqq + sqk) - 2.0 * dot

    rv = rc = None
    for g in range(nchunks // _BATCH):
        bv = [d2[:, (g * _BATCH + j) * _CHUNK:(g * _BATCH + j + 1) * _CHUNK]
              for j in range(_BATCH)]
        bc = [jnp.full((rows, _CHUNK), g * _BATCH + j, jnp.int32)
              for j in range(_BATCH)]
        bv, bc = _sort_batch(bv, bc)
        if rv is None:
            rv, rc = bv, bc
        else:
            rv, rc = _merge_keep_low(rv, rc, bv, bc)

    lane = jax.lax.broadcasted_iota(jnp.int32, (rows, _CHUNK), 1)
    rf = [ci * _CHUNK + lane for ci in rc]
    big = jnp.int32(npad)
    vals = []
    idxs = []
    for _ in range(_K):
        head_v = rv[0]
        head_f = rf[0]
        m = jnp.min(head_v, axis=1, keepdims=True)
        elig = head_v == m
        fmin = jnp.min(jnp.where(elig, head_f, big), axis=1, keepdims=True)
        vals.append(m)
        idxs.append(fmin)
        win = elig & (head_f == fmin)
        for s in range(_K - 1):
            rv[s] = jnp.where(win, rv[s + 1], rv[s])
            rf[s] = jnp.where(win, rf[s + 1], rf[s])

    v = jnp.concatenate(vals, axis=1)
    i = jnp.concatenate(idxs, axis=1)
    idx_ref[...] = i
    dist_ref[...] = jnp.sqrt(jnp.maximum(v, 0.0))
    row0 = pl.program_id(0) * rows
    dst_ref[...] = row0 + jax.lax.broadcasted_iota(jnp.int32, (rows, _K), 0)


def kernel(positions, numbers):
    n = positions.shape[0]
    span = _BATCH * _CHUNK
    npad = ((n + span - 1) // span) * span
    grid = npad // _ROWS

    pos_pad = jnp.zeros((npad, 128), jnp.float32).at[:n, :3].set(positions)
    kpos = jnp.zeros((128, npad), jnp.float32).at[:3, :n].set(positions.T)

    idx, dst, dists = pl.pallas_call(
        functools.partial(_knn_tile_kernel, n),
        grid=(grid,),
        in_specs=[
            pl.BlockSpec((_ROWS, 128), lambda i: (i, 0)),
            pl.BlockSpec((128, npad), lambda i: (0, 0)),
        ],
        out_specs=[
            pl.BlockSpec((_ROWS, _K), lambda i: (i, 0)),
            pl.BlockSpec((_ROWS, _K), lambda i: (i, 0)),
            pl.BlockSpec((_ROWS, _K), lambda i: (i, 0)),
        ],
        out_shape=[
            jax.ShapeDtypeStruct((npad, _K), jnp.int32),
            jax.ShapeDtypeStruct((npad, _K), jnp.int32),
            jax.ShapeDtypeStruct((npad, _K), jnp.float32),
        ],
    )(pos_pad, kpos)

    src = idx[:n].reshape(-1)
    dst = dst[:n].reshape(-1)
    return (src, dst, numbers, dists[:n])
